# Initial kernel scaffold; baseline (speedup 1.0000x reference)
import functools

import jax
import jax.numpy as jnp
from jax import lax
from jax.experimental import pallas as pl
from jax.experimental.pallas import tpu as pltpu
from jax.experimental.pallas import tpu_sc as plsc

_N = 10000
_E = 320000
_H = 128
_DE = 16
_DIN = 2 * _H + _DE

_NC = 2
_NS = 16
_NW = _NC * _NS
_EPW = _E // _NW
_CH = 80
_NIT = _EPW // _CH

_NPT = _N // _NS
_NPB = 125
_NOB = _NPT // _NPB


def _wid():
    return lax.axis_index("c") * _NS + lax.axis_index("s")


def _gather_body(z_hbm, src_hbm, dst_hbm, zsrc_hbm, zdst_hbm,
                 idx_s, idx_d, rows_s, rows_d, sem_s, sem_d):
    wid = _wid()
    base = wid * _EPW
    pltpu.sync_copy(src_hbm.at[wid], idx_s)
    pltpu.sync_copy(dst_hbm.at[wid], idx_d)

    def body(i, carry):
        cs = pltpu.async_copy(z_hbm.at[idx_s.at[i]], rows_s, sem_s)
        cd = pltpu.async_copy(z_hbm.at[idx_d.at[i]], rows_d, sem_d)
        cs.wait()
        cd.wait()
        pltpu.sync_copy(rows_s, zsrc_hbm.at[pl.ds(base + i * _CH, _CH)])
        pltpu.sync_copy(rows_d, zdst_hbm.at[pl.ds(base + i * _CH, _CH)])
        return carry

    lax.fori_loop(0, _NIT, body, 0)


def _sc_gather(z, src3, dst3):
    mesh = plsc.VectorSubcoreMesh(core_axis_name="c", subcore_axis_name="s",
                                  num_cores=_NC)
    k = functools.partial(
        pl.kernel,
        out_type=(jax.ShapeDtypeStruct((_E, _H), jnp.float32),
                  jax.ShapeDtypeStruct((_E, _H), jnp.float32)),
        mesh=mesh,
        scratch_types=[
            pltpu.VMEM((_NIT, _CH), jnp.int32),
            pltpu.VMEM((_NIT, _CH), jnp.int32),
            pltpu.VMEM((_CH, _H), jnp.float32),
            pltpu.VMEM((_CH, _H), jnp.float32),
            pltpu.SemaphoreType.DMA,
            pltpu.SemaphoreType.DMA,
        ],
    )(_gather_body)
    return k(z, src3, dst3)


def _dense_body(zs_ref, zd_ref, ea_ref, w1_ref, w2_ref, w3_ref, bffw_ref,
                wf1_ref, wf2_ref, bf_ref, ws1_ref, ws2_ref, bs_ref,
                zedge_ref, msg_ref):
    zs = zs_ref[...]
    zd = zd_ref[...]
    ea = ea_ref[...]
    dot = functools.partial(jnp.dot, preferred_element_type=jnp.float32)
    pe = (dot(zs, w1_ref[...]) + dot(zd, w2_ref[...]) + dot(ea, w3_ref[...])
          + bffw_ref[...])
    zedge_ref[...] = jnp.maximum(pe, 0.0)
    gp = dot(zd, wf1_ref[...]) + dot(zs, wf2_ref[...]) + bf_ref[...]
    cp = dot(zd, ws1_ref[...]) + dot(zs, ws2_ref[...]) + bs_ref[...]
    gate = 1.0 / (1.0 + jnp.exp(-gp))
    core = jnp.maximum(cp, 0.0) + jnp.log(1.0 + jnp.exp(-jnp.abs(cp)))
    msg_ref[...] = gate * core


def _tc_dense(zsrc, zdst, edge_attr, w1, w2, w3, bffw, wf1, wf2, bf, ws1, ws2, bs):
    be = 2000
    grid = (_E // be,)
    row = lambda i: (i, 0)
    rep = lambda i: (0, 0)
    return pl.pallas_call(
        _dense_body,
        grid=grid,
        in_specs=[
            pl.BlockSpec((be, _H), row),
            pl.BlockSpec((be, _H), row),
            pl.BlockSpec((be, _DE), row),
            pl.BlockSpec((_H, _DIN), rep),
            pl.BlockSpec((_H, _DIN), rep),
            pl.BlockSpec((_DE, _DIN), rep),
            pl.BlockSpec((1, _DIN), rep),
            pl.BlockSpec((_H, _H), rep),
            pl.BlockSpec((_H, _H), rep),
            pl.BlockSpec((1, _H), rep),
            pl.BlockSpec((_H, _H), rep),
            pl.BlockSpec((_H, _H), rep),
            pl.BlockSpec((1, _H), rep),
        ],
        out_specs=[
            pl.BlockSpec((be, _DIN), row),
            pl.BlockSpec((be, _H), row),
        ],
        out_shape=[
            jax.ShapeDtypeStruct((_E, _DIN), jnp.float32),
            jax.ShapeDtypeStruct((_E, _H), jnp.float32),
        ],
    )(zsrc, zdst, edge_attr, w1, w2, w3, bffw, wf1, wf2, bf, ws1, ws2, bs)


def _scatter_body(msg_hbm, dst_hbm, part_hbm, idx_d, rows, stage, agg_sh, sem):
    c = lax.axis_index("c")
    s = lax.axis_index("s")
    wid = c * _NS + s
    base = wid * _EPW

    def zr(r, carry):
        def zk(k, carry2):
            stage[r, pl.ds(k * 16, 16)] = jnp.zeros((16,), jnp.float32)
            return carry2
        return lax.fori_loop(0, _H // 16, zk, carry)

    lax.fori_loop(0, _NPB, zr, 0)

    def zcopy(j, carry):
        pltpu.sync_copy(stage, agg_sh.at[pl.ds(s * _NPT + j * _NPB, _NPB)])
        return carry

    lax.fori_loop(0, _NOB, zcopy, 0)
    pltpu.sync_copy(dst_hbm.at[wid], idx_d)
    plsc.subcore_barrier()

    def body(i, carry):
        cm = pltpu.async_copy(msg_hbm.at[pl.ds(base + i * _CH, _CH)], rows, sem)
        cm.wait()
        pltpu.sync_copy(rows, agg_sh.at[idx_d.at[i]], add=True)
        return carry

    lax.fori_loop(0, _NIT, body, 0)
    plsc.subcore_barrier()

    def ocopy(j, carry):
        r0 = s * _NPT + j * _NPB
        pltpu.sync_copy(agg_sh.at[pl.ds(r0, _NPB)], stage)
        pltpu.sync_copy(stage, part_hbm.at[c, pl.ds(r0, _NPB)])
        return carry

    lax.fori_loop(0, _NOB, ocopy, 0)


def _sc_scatter(msg, dst3):
    mesh = plsc.VectorSubcoreMesh(core_axis_name="c", subcore_axis_name="s",
                                  num_cores=_NC)
    k = functools.partial(
        pl.kernel,
        out_type=jax.ShapeDtypeStruct((_NC, _N, _H), jnp.float32),
        mesh=mesh,
        scratch_types=[
            pltpu.VMEM((_NIT, _CH), jnp.int32),
            pltpu.VMEM((_CH, _H), jnp.float32),
            pltpu.VMEM((_NPB, _H), jnp.float32),
            pltpu.VMEM_SHARED((_N, _H), jnp.float32),
            pltpu.SemaphoreType.DMA,
        ],
    )(_scatter_body)
    return k(msg, dst3)


def _combine_body(z_ref, p_ref, out_ref):
    out_ref[...] = z_ref[...] + p_ref[0] + p_ref[1]


def _tc_combine(z, part):
    bn = 2000
    return pl.pallas_call(
        _combine_body,
        grid=(_N // bn,),
        in_specs=[
            pl.BlockSpec((bn, _H), lambda i: (i, 0)),
            pl.BlockSpec((_NC, bn, _H), lambda i: (0, i, 0)),
        ],
        out_specs=pl.BlockSpec((bn, _H), lambda i: (i, 0)),
        out_shape=jax.ShapeDtypeStruct((_N, _H), jnp.float32),
    )(z, part)


def kernel(z, edge_attr, edge_index, W_ffw, b_ffw, W_f, b_f, W_s, b_s):
    src3 = edge_index[0].reshape(_NW, _NIT, _CH)
    dst3 = edge_index[1].reshape(_NW, _NIT, _CH)

    zsrc, zdst = _sc_gather(z, src3, dst3)

    w1 = W_ffw[:_H]
    w2 = W_ffw[_H:2 * _H]
    w3 = W_ffw[2 * _H:]
    wf1 = W_f[:_H]
    wf2 = W_f[_H:]
    ws1 = W_s[:_H]
    ws2 = W_s[_H:]
    z_edge, msg = _tc_dense(zsrc, zdst, edge_attr,
                            w1, w2, w3, b_ffw.reshape(1, _DIN),
                            wf1, wf2, b_f.reshape(1, _H),
                            ws1, ws2, b_s.reshape(1, _H))

    part = _sc_scatter(msg, dst3)
    z_node = _tc_combine(z, part)
    return (z_node, z_edge)


# speedup vs baseline: 2.4947x; 2.4947x over previous
import functools

import jax
import jax.numpy as jnp
from jax import lax
from jax.experimental import pallas as pl
from jax.experimental.pallas import tpu as pltpu
from jax.experimental.pallas import tpu_sc as plsc

_N = 10000
_E = 320000
_H = 128
_DE = 16
_DIN = 2 * _H + _DE

_NC = 2
_NS = 16
_NW = _NC * _NS
_EPW = _E // _NW
_CH = 80
_NIT = _EPW // _CH

_NAGG = 10240
_NPT = _NAGG // _NS
_NPB = 128
_NOB = _NPT // _NPB


def _wid():
    return lax.axis_index("c") * _NS + lax.axis_index("s")


def _gather_body(z_hbm, src_hbm, dst_hbm, zsrc_hbm, zdst_hbm,
                 idx_s, idx_d, rows_s, rows_d, sem_s, sem_d):
    wid = _wid()
    base = wid * _EPW
    pltpu.sync_copy(src_hbm.at[wid], idx_s)
    pltpu.sync_copy(dst_hbm.at[wid], idx_d)

    def body(i, carry):
        cs = pltpu.async_copy(z_hbm.at[idx_s.at[i]], rows_s, sem_s)
        cd = pltpu.async_copy(z_hbm.at[idx_d.at[i]], rows_d, sem_d)
        cs.wait()
        cd.wait()
        pltpu.sync_copy(rows_s, zsrc_hbm.at[pl.ds(base + i * _CH, _CH)])
        pltpu.sync_copy(rows_d, zdst_hbm.at[pl.ds(base + i * _CH, _CH)])
        return carry

    lax.fori_loop(0, _NIT, body, 0)


def _sc_gather(z, src3, dst3):
    mesh = plsc.VectorSubcoreMesh(core_axis_name="c", subcore_axis_name="s",
                                  num_cores=_NC)
    k = functools.partial(
        pl.kernel,
        out_type=(jax.ShapeDtypeStruct((_E, _H), jnp.float32),
                  jax.ShapeDtypeStruct((_E, _H), jnp.float32)),
        mesh=mesh,
        scratch_types=[
            pltpu.VMEM((_NIT, _CH), jnp.int32),
            pltpu.VMEM((_NIT, _CH), jnp.int32),
            pltpu.VMEM((_CH, _H), jnp.float32),
            pltpu.VMEM((_CH, _H), jnp.float32),
            pltpu.SemaphoreType.DMA,
            pltpu.SemaphoreType.DMA,
        ],
    )(_gather_body)
    return k(z, src3, dst3)


def _dense_body(zs_ref, zd_ref, ea_ref, w1_ref, w2_ref, w3_ref, bffw_ref,
                wf1_ref, wf2_ref, bf_ref, ws1_ref, ws2_ref, bs_ref,
                zedge_ref, msg_ref):
    zs = zs_ref[...]
    zd = zd_ref[...]
    ea = ea_ref[...]
    dot = functools.partial(jnp.dot, preferred_element_type=jnp.float32)
    pe = (dot(zs, w1_ref[...]) + dot(zd, w2_ref[...]) + dot(ea, w3_ref[...])
          + bffw_ref[...])
    zedge_ref[...] = jnp.maximum(pe, 0.0)
    gp = dot(zd, wf1_ref[...]) + dot(zs, wf2_ref[...]) + bf_ref[...]
    cp = dot(zd, ws1_ref[...]) + dot(zs, ws2_ref[...]) + bs_ref[...]
    gate = 1.0 / (1.0 + jnp.exp(-gp))
    core = jnp.maximum(cp, 0.0) + jnp.log(1.0 + jnp.exp(-jnp.abs(cp)))
    msg_ref[...] = gate * core


def _tc_dense(zsrc, zdst, edge_attr, w1, w2, w3, bffw, wf1, wf2, bf, ws1, ws2, bs):
    be = 2000
    grid = (_E // be,)
    row = lambda i: (i, 0)
    rep = lambda i: (0, 0)
    return pl.pallas_call(
        _dense_body,
        grid=grid,
        in_specs=[
            pl.BlockSpec((be, _H), row),
            pl.BlockSpec((be, _H), row),
            pl.BlockSpec((be, _DE), row),
            pl.BlockSpec((_H, _DIN), rep),
            pl.BlockSpec((_H, _DIN), rep),
            pl.BlockSpec((_DE, _DIN), rep),
            pl.BlockSpec((1, _DIN), rep),
            pl.BlockSpec((_H, _H), rep),
            pl.BlockSpec((_H, _H), rep),
            pl.BlockSpec((1, _H), rep),
            pl.BlockSpec((_H, _H), rep),
            pl.BlockSpec((_H, _H), rep),
            pl.BlockSpec((1, _H), rep),
        ],
        out_specs=[
            pl.BlockSpec((be, _DIN), row),
            pl.BlockSpec((be, _H), row),
        ],
        out_shape=[
            jax.ShapeDtypeStruct((_E, _DIN), jnp.float32),
            jax.ShapeDtypeStruct((_E, _H), jnp.float32),
        ],
    )(zsrc, zdst, edge_attr, w1, w2, w3, bffw, wf1, wf2, bf, ws1, ws2, bs)


def _scatter_body(msg_hbm, dst_hbm, part_hbm, idx_d, rows, stage, agg_sh, sem):
    c = lax.axis_index("c")
    s = lax.axis_index("s")
    wid = c * _NS + s
    base = wid * _EPW

    def zr(r, carry):
        def zk(k, carry2):
            stage[r, pl.ds(k * 16, 16)] = jnp.zeros((16,), jnp.float32)
            return carry2
        return lax.fori_loop(0, _H // 16, zk, carry)

    lax.fori_loop(0, _NPB, zr, 0)

    def zcopy(j, carry):
        pltpu.sync_copy(stage, agg_sh.at[pl.ds(s * _NPT + j * _NPB, _NPB)])
        return carry

    lax.fori_loop(0, _NOB, zcopy, 0)
    pltpu.sync_copy(dst_hbm.at[wid], idx_d)
    plsc.subcore_barrier()

    def body(i, carry):
        cm = pltpu.async_copy(msg_hbm.at[pl.ds(base + i * _CH, _CH)], rows, sem)
        cm.wait()
        pltpu.sync_copy(rows, agg_sh.at[idx_d.at[i]], add=True)
        return carry

    lax.fori_loop(0, _NIT, body, 0)
    plsc.subcore_barrier()

    def ocopy(j, carry):
        r0 = s * _NPT + j * _NPB
        pltpu.sync_copy(agg_sh.at[pl.ds(r0, _NPB)], stage)
        pltpu.sync_copy(stage, part_hbm.at[c, pl.ds(r0, _NPB)])
        return carry

    lax.fori_loop(0, _NOB, ocopy, 0)


def _sc_scatter(msg, dst3):
    mesh = plsc.VectorSubcoreMesh(core_axis_name="c", subcore_axis_name="s",
                                  num_cores=_NC)
    k = functools.partial(
        pl.kernel,
        out_type=jax.ShapeDtypeStruct((_NC, _NAGG, _H), jnp.float32),
        mesh=mesh,
        scratch_types=[
            pltpu.VMEM((_NIT, _CH), jnp.int32),
            pltpu.VMEM((_CH, _H), jnp.float32),
            pltpu.VMEM((_NPB, _H), jnp.float32),
            pltpu.VMEM_SHARED((_NAGG, _H), jnp.float32),
            pltpu.SemaphoreType.DMA,
        ],
    )(_scatter_body)
    return k(msg, dst3)


def _combine_body(z_ref, p_ref, out_ref):
    out_ref[...] = z_ref[...] + p_ref[0] + p_ref[1]


def _tc_combine(z, part):
    bn = 2000
    return pl.pallas_call(
        _combine_body,
        grid=(_N // bn,),
        in_specs=[
            pl.BlockSpec((bn, _H), lambda i: (i, 0)),
            pl.BlockSpec((_NC, bn, _H), lambda i: (0, i, 0)),
        ],
        out_specs=pl.BlockSpec((bn, _H), lambda i: (i, 0)),
        out_shape=jax.ShapeDtypeStruct((_N, _H), jnp.float32),
    )(z, part)


def kernel(z, edge_attr, edge_index, W_ffw, b_ffw, W_f, b_f, W_s, b_s):
    src3 = edge_index[0].reshape(_NW, _NIT, _CH)
    dst3 = edge_index[1].reshape(_NW, _NIT, _CH)

    zsrc, zdst = _sc_gather(z, src3, dst3)

    w1 = W_ffw[:_H]
    w2 = W_ffw[_H:2 * _H]
    w3 = W_ffw[2 * _H:]
    wf1 = W_f[:_H]
    wf2 = W_f[_H:]
    ws1 = W_s[:_H]
    ws2 = W_s[_H:]
    z_edge, msg = _tc_dense(zsrc, zdst, edge_attr,
                            w1, w2, w3, b_ffw.reshape(1, _DIN),
                            wf1, wf2, b_f.reshape(1, _H),
                            ws1, ws2, b_s.reshape(1, _H))

    part = _sc_scatter(msg, dst3)
    z_node = _tc_combine(z, part)
    return (z_node, z_edge)
